# Initial kernel scaffold; baseline (speedup 1.0000x reference)
import jax
import jax.numpy as jnp
from jax.experimental import pallas as pl


def kernel(inputs, bias_mat, training, W1, a_src1, b_src1, a_dst1, b_dst1, bias1, W2, a_src2, b_src2, a_dst2, b_dst2, bias2):
    raise NotImplementedError("write your pallas kernel here")


# speedup vs baseline: 1.9669x; 1.9669x over previous
import functools

import jax
import jax.numpy as jnp
from jax.experimental import pallas as pl


def _proj_body(x_ref, w_ref, asrc_ref, adst_ref, bs_ref, bd_ref,
               fts_ref, f1_ref, f2_ref):
    fts = jnp.dot(x_ref[...], w_ref[...], preferred_element_type=jnp.float32)
    fts_ref[...] = fts
    f1_ref[...] = jnp.dot(fts, asrc_ref[...],
                          preferred_element_type=jnp.float32) + bs_ref[...]
    f2_ref[...] = jnp.dot(fts, adst_ref[...],
                          preferred_element_type=jnp.float32) + bd_ref[...]


def _project(x, w_cat, a_src_cat, a_dst_cat, b_src_row, b_dst_row, row_block):
    n, fin = x.shape
    dtot = w_cat.shape[1]
    hh = a_src_cat.shape[1]
    grid = (n // row_block,)
    return pl.pallas_call(
        _proj_body,
        grid=grid,
        in_specs=[
            pl.BlockSpec((row_block, fin), lambda i: (i, 0)),
            pl.BlockSpec((fin, dtot), lambda i: (0, 0)),
            pl.BlockSpec((dtot, hh), lambda i: (0, 0)),
            pl.BlockSpec((dtot, hh), lambda i: (0, 0)),
            pl.BlockSpec((1, hh), lambda i: (0, 0)),
            pl.BlockSpec((1, hh), lambda i: (0, 0)),
        ],
        out_specs=[
            pl.BlockSpec((row_block, dtot), lambda i: (i, 0)),
            pl.BlockSpec((row_block, hh), lambda i: (i, 0)),
            pl.BlockSpec((row_block, hh), lambda i: (i, 0)),
        ],
        out_shape=[
            jax.ShapeDtypeStruct((n, dtot), jnp.float32),
            jax.ShapeDtypeStruct((n, hh), jnp.float32),
            jax.ShapeDtypeStruct((n, hh), jnp.float32),
        ],
    )(x, w_cat, a_src_cat, a_dst_cat, b_src_row, b_dst_row)


def _attn_body(bias_ref, f1_ref, f2t_ref, fts_ref, bout_ref, out_ref,
               *, heads, d, elu):
    bias = bias_ref[...]
    for h in range(heads):
        logit = f1_ref[:, h][:, None] + f2t_ref[h, :][None, :]
        logit = jnp.where(logit > 0, logit, 0.2 * logit)
        logit = logit + bias
        m = jnp.max(logit, axis=1, keepdims=True)
        e = jnp.exp(logit - m)
        s = jnp.sum(e, axis=1, keepdims=True)
        coefs = e / s
        v = jnp.dot(coefs, fts_ref[:, h * d:(h + 1) * d],
                    preferred_element_type=jnp.float32)
        v = v + bout_ref[:, h * d:(h + 1) * d]
        if elu:
            v = jnp.where(v > 0, v, jnp.exp(jnp.minimum(v, 0.0)) - 1.0)
        out_ref[:, h * d:(h + 1) * d] = v


def _attn_layer(bias2d, f1, f2t, fts, b_out_row, heads, d, elu, row_block):
    n = bias2d.shape[0]
    grid = (n // row_block,)
    body = functools.partial(_attn_body, heads=heads, d=d, elu=elu)
    return pl.pallas_call(
        body,
        grid=grid,
        in_specs=[
            pl.BlockSpec((row_block, n), lambda i: (i, 0)),
            pl.BlockSpec((row_block, heads), lambda i: (i, 0)),
            pl.BlockSpec((heads, n), lambda i: (0, 0)),
            pl.BlockSpec((n, heads * d), lambda i: (0, 0)),
            pl.BlockSpec((1, heads * d), lambda i: (0, 0)),
        ],
        out_specs=pl.BlockSpec((row_block, heads * d), lambda i: (i, 0)),
        out_shape=jax.ShapeDtypeStruct((n, heads * d), jnp.float32),
    )(bias2d, f1, f2t, fts, b_out_row)


def kernel(inputs, bias_mat, training, W1, a_src1, b_src1, a_dst1, b_dst1,
           bias1, W2, a_src2, b_src2, a_dst2, b_dst2, bias2):
    n = inputs.shape[1]
    f_in = inputs.shape[2]
    heads1, _, h_dim = W1.shape
    c_dim = W2.shape[1]

    x = inputs.reshape(n, f_in)
    bias2d = bias_mat.reshape(n, n)
    rb_proj = 2000 if n % 2000 == 0 else n
    rb_attn = 200 if n % 200 == 0 else n

    w1_cat = jnp.transpose(W1, (1, 0, 2)).reshape(f_in, heads1 * h_dim)
    eye = jnp.eye(heads1, dtype=jnp.float32)
    a_src1_cat = (a_src1[:, :, 0][:, :, None] * eye[:, None, :]) \
        .reshape(heads1 * h_dim, heads1)
    a_dst1_cat = (a_dst1[:, :, 0][:, :, None] * eye[:, None, :]) \
        .reshape(heads1 * h_dim, heads1)

    fts1, f1_1, f2_1 = _project(x, w1_cat, a_src1_cat, a_dst1_cat,
                                b_src1.reshape(1, heads1),
                                b_dst1.reshape(1, heads1), row_block=rb_proj)
    h1 = _attn_layer(bias2d, f1_1, f2_1.T, fts1,
                     bias1.reshape(1, heads1 * h_dim),
                     heads1, h_dim, elu=True, row_block=rb_attn)

    fts2, f1_2, f2_2 = _project(h1, W2, a_src2, a_dst2,
                                b_src2.reshape(1, 1), b_dst2.reshape(1, 1),
                                row_block=rb_proj)
    out = _attn_layer(bias2d, f1_2, f2_2.T, fts2, bias2.reshape(1, c_dim),
                      1, c_dim, elu=False, row_block=rb_attn)
    return out.reshape(1, n, c_dim)


# speedup vs baseline: 2.6492x; 1.3469x over previous
import functools

import jax
import jax.numpy as jnp
from jax.experimental import pallas as pl


def _proj_body(x_ref, w_ref, asrc_ref, adst_ref, bs_ref, bd_ref,
               fts_ref, f1_ref, f2_ref):
    fts = jnp.dot(x_ref[...], w_ref[...], preferred_element_type=jnp.float32)
    fts_ref[...] = fts
    f1_ref[...] = jnp.dot(fts, asrc_ref[...],
                          preferred_element_type=jnp.float32) + bs_ref[...]
    f2_ref[...] = jnp.dot(fts, adst_ref[...],
                          preferred_element_type=jnp.float32) + bd_ref[...]


def _project(x, w_cat, a_src_cat, a_dst_cat, b_src_row, b_dst_row, row_block):
    n, fin = x.shape
    dtot = w_cat.shape[1]
    hh = a_src_cat.shape[1]
    grid = (n // row_block,)
    return pl.pallas_call(
        _proj_body,
        grid=grid,
        in_specs=[
            pl.BlockSpec((row_block, fin), lambda i: (i, 0)),
            pl.BlockSpec((fin, dtot), lambda i: (0, 0)),
            pl.BlockSpec((dtot, hh), lambda i: (0, 0)),
            pl.BlockSpec((dtot, hh), lambda i: (0, 0)),
            pl.BlockSpec((1, hh), lambda i: (0, 0)),
            pl.BlockSpec((1, hh), lambda i: (0, 0)),
        ],
        out_specs=[
            pl.BlockSpec((row_block, dtot), lambda i: (i, 0)),
            pl.BlockSpec((row_block, hh), lambda i: (i, 0)),
            pl.BlockSpec((row_block, hh), lambda i: (i, 0)),
        ],
        out_shape=[
            jax.ShapeDtypeStruct((n, dtot), jnp.float32),
            jax.ShapeDtypeStruct((n, hh), jnp.float32),
            jax.ShapeDtypeStruct((n, hh), jnp.float32),
        ],
    )(x, w_cat, a_src_cat, a_dst_cat, b_src_row, b_dst_row)


def _attn_body(bias_ref, f1_ref, f2t_ref, fts_ref, bout_ref, out_ref,
               *, heads, d, elu):
    edge = bias_ref[...] >= -0.5
    for h in range(heads):
        f1 = f1_ref[:, h][:, None]
        f2 = f2t_ref[h, :][None, :]
        u, up = jnp.exp(f1), jnp.exp(0.2 * f1)
        v, vp = jnp.exp(f2), jnp.exp(0.2 * f2)
        e = jnp.maximum(u * v, up * vp)
        e = jnp.where(edge, e, 0.0)
        s = jnp.sum(e, axis=1, keepdims=True)
        num = jnp.dot(e, fts_ref[:, h * d:(h + 1) * d],
                      preferred_element_type=jnp.float32)
        o = num / s + bout_ref[:, h * d:(h + 1) * d]
        if elu:
            o = jnp.where(o > 0, o, jnp.exp(jnp.minimum(o, 0.0)) - 1.0)
        out_ref[:, h * d:(h + 1) * d] = o


def _attn_layer(bias2d, f1, f2t, fts, b_out_row, heads, d, elu, row_block):
    n = bias2d.shape[0]
    grid = (n // row_block,)
    body = functools.partial(_attn_body, heads=heads, d=d, elu=elu)
    return pl.pallas_call(
        body,
        grid=grid,
        in_specs=[
            pl.BlockSpec((row_block, n), lambda i: (i, 0)),
            pl.BlockSpec((row_block, heads), lambda i: (i, 0)),
            pl.BlockSpec((heads, n), lambda i: (0, 0)),
            pl.BlockSpec((n, heads * d), lambda i: (0, 0)),
            pl.BlockSpec((1, heads * d), lambda i: (0, 0)),
        ],
        out_specs=pl.BlockSpec((row_block, heads * d), lambda i: (i, 0)),
        out_shape=jax.ShapeDtypeStruct((n, heads * d), jnp.float32),
    )(bias2d, f1, f2t, fts, b_out_row)


def kernel(inputs, bias_mat, training, W1, a_src1, b_src1, a_dst1, b_dst1,
           bias1, W2, a_src2, b_src2, a_dst2, b_dst2, bias2):
    n = inputs.shape[1]
    f_in = inputs.shape[2]
    heads1, _, h_dim = W1.shape
    c_dim = W2.shape[1]

    x = inputs.reshape(n, f_in)
    bias2d = bias_mat.reshape(n, n)
    rb_proj = 2000 if n % 2000 == 0 else n
    rb_attn = 200 if n % 200 == 0 else n

    w1_cat = jnp.transpose(W1, (1, 0, 2)).reshape(f_in, heads1 * h_dim)
    eye = jnp.eye(heads1, dtype=jnp.float32)
    a_src1_cat = (a_src1[:, :, 0][:, :, None] * eye[:, None, :]) \
        .reshape(heads1 * h_dim, heads1)
    a_dst1_cat = (a_dst1[:, :, 0][:, :, None] * eye[:, None, :]) \
        .reshape(heads1 * h_dim, heads1)

    fts1, f1_1, f2_1 = _project(x, w1_cat, a_src1_cat, a_dst1_cat,
                                b_src1.reshape(1, heads1),
                                b_dst1.reshape(1, heads1), row_block=rb_proj)
    h1 = _attn_layer(bias2d, f1_1, f2_1.T, fts1,
                     bias1.reshape(1, heads1 * h_dim),
                     heads1, h_dim, elu=True, row_block=rb_attn)

    fts2, f1_2, f2_2 = _project(h1, W2, a_src2, a_dst2,
                                b_src2.reshape(1, 1), b_dst2.reshape(1, 1),
                                row_block=rb_proj)
    out = _attn_layer(bias2d, f1_2, f2_2.T, fts2, bias2.reshape(1, c_dim),
                      1, c_dim, elu=False, row_block=rb_attn)
    return out.reshape(1, n, c_dim)


# speedup vs baseline: 3.4819x; 1.3143x over previous
import functools

import jax
import jax.numpy as jnp
from jax.experimental import pallas as pl

_SEG = 128


def _proj_body(x_ref, w_ref, asrc_ref, adst_ref, bs_ref, bd_ref,
               fts_ref, f1_ref, f2_ref, *, heads, d):
    fts = jnp.dot(x_ref[...], w_ref[...], preferred_element_type=jnp.float32)
    f1_ref[...] = jnp.dot(fts, asrc_ref[...],
                          preferred_element_type=jnp.float32) + bs_ref[...]
    f2_ref[...] = jnp.dot(fts, adst_ref[...],
                          preferred_element_type=jnp.float32) + bd_ref[...]
    fts_ref[...] = fts
    ones = jnp.ones((fts.shape[0], 1), jnp.float32)
    for h in range(heads):
        fts_ref[:, h * _SEG + d:h * _SEG + d + 1] = ones


def _project(x, w_cat, a_src_cat, a_dst_cat, b_src_row, b_dst_row,
             heads, d, row_block):
    n, fin = x.shape
    dtot = w_cat.shape[1]
    hh = a_src_cat.shape[1]
    grid = (n // row_block,)
    body = functools.partial(_proj_body, heads=heads, d=d)
    return pl.pallas_call(
        body,
        grid=grid,
        in_specs=[
            pl.BlockSpec((row_block, fin), lambda i: (i, 0)),
            pl.BlockSpec((fin, dtot), lambda i: (0, 0)),
            pl.BlockSpec((dtot, hh), lambda i: (0, 0)),
            pl.BlockSpec((dtot, hh), lambda i: (0, 0)),
            pl.BlockSpec((1, hh), lambda i: (0, 0)),
            pl.BlockSpec((1, hh), lambda i: (0, 0)),
        ],
        out_specs=[
            pl.BlockSpec((row_block, dtot), lambda i: (i, 0)),
            pl.BlockSpec((row_block, hh), lambda i: (i, 0)),
            pl.BlockSpec((row_block, hh), lambda i: (i, 0)),
        ],
        out_shape=[
            jax.ShapeDtypeStruct((n, dtot), jnp.float32),
            jax.ShapeDtypeStruct((n, hh), jnp.float32),
            jax.ShapeDtypeStruct((n, hh), jnp.float32),
        ],
    )(x, w_cat, a_src_cat, a_dst_cat, b_src_row, b_dst_row)


def _attn_body(bias_ref, f1_ref, f2t_ref, fts_ref, bout_ref, out_ref,
               *, heads, d, elu):
    eb = jnp.exp(bias_ref[...])
    for h in range(heads):
        f1 = f1_ref[:, h][:, None]
        f2 = f2t_ref[h, :][None, :]
        u, up = jnp.exp(f1), jnp.exp(0.2 * f1)
        v, vp = jnp.exp(f2), jnp.exp(0.2 * f2)
        e = jnp.maximum(u * v, up * vp) * eb
        num = jnp.dot(e, fts_ref[:, h * _SEG:(h + 1) * _SEG],
                      preferred_element_type=jnp.float32)
        o = num[:, :d] / num[:, d:d + 1] + bout_ref[:, h * d:(h + 1) * d]
        if elu:
            o = jnp.where(o > 0, o, jnp.exp(jnp.minimum(o, 0.0)) - 1.0)
        out_ref[:, h * d:(h + 1) * d] = o


def _attn_layer(bias2d, f1, f2t, fts, b_out_row, heads, d, elu, row_block):
    n = bias2d.shape[0]
    grid = (n // row_block,)
    body = functools.partial(_attn_body, heads=heads, d=d, elu=elu)
    return pl.pallas_call(
        body,
        grid=grid,
        in_specs=[
            pl.BlockSpec((row_block, n), lambda i: (i, 0)),
            pl.BlockSpec((row_block, heads), lambda i: (i, 0)),
            pl.BlockSpec((heads, n), lambda i: (0, 0)),
            pl.BlockSpec((n, heads * _SEG), lambda i: (0, 0)),
            pl.BlockSpec((1, heads * d), lambda i: (0, 0)),
        ],
        out_specs=pl.BlockSpec((row_block, heads * d), lambda i: (i, 0)),
        out_shape=jax.ShapeDtypeStruct((n, heads * d), jnp.float32),
    )(bias2d, f1, f2t, fts, b_out_row)


def _pad_params(W_heads, a_src_heads, a_dst_heads, d):
    heads, fin, _ = W_heads.shape
    w_cat = jnp.zeros((fin, heads * _SEG), jnp.float32)
    a_src = jnp.zeros((heads * _SEG, heads), jnp.float32)
    a_dst = jnp.zeros((heads * _SEG, heads), jnp.float32)
    for h in range(heads):
        w_cat = w_cat.at[:, h * _SEG:h * _SEG + d].set(W_heads[h])
        a_src = a_src.at[h * _SEG:h * _SEG + d, h].set(a_src_heads[h, :, 0])
        a_dst = a_dst.at[h * _SEG:h * _SEG + d, h].set(a_dst_heads[h, :, 0])
    return w_cat, a_src, a_dst


def kernel(inputs, bias_mat, training, W1, a_src1, b_src1, a_dst1, b_dst1,
           bias1, W2, a_src2, b_src2, a_dst2, b_dst2, bias2):
    n = inputs.shape[1]
    f_in = inputs.shape[2]
    heads1, _, h_dim = W1.shape
    c_dim = W2.shape[1]

    x = inputs.reshape(n, f_in)
    bias2d = bias_mat.reshape(n, n)
    rb_proj = 2000 if n % 2000 == 0 else n
    rb_attn = 200 if n % 200 == 0 else n

    w1_cat, a_src1_cat, a_dst1_cat = _pad_params(W1, a_src1, a_dst1, h_dim)
    fts1, f1_1, f2_1 = _project(x, w1_cat, a_src1_cat, a_dst1_cat,
                                b_src1.reshape(1, heads1),
                                b_dst1.reshape(1, heads1),
                                heads1, h_dim, rb_proj)
    h1 = _attn_layer(bias2d, f1_1, f2_1.T, fts1,
                     bias1.reshape(1, heads1 * h_dim),
                     heads1, h_dim, elu=True, row_block=rb_attn)

    w2_cat, a_src2_cat, a_dst2_cat = _pad_params(
        W2[None], a_src2[None], a_dst2[None], c_dim)
    fts2, f1_2, f2_2 = _project(h1, w2_cat, a_src2_cat, a_dst2_cat,
                                b_src2.reshape(1, 1), b_dst2.reshape(1, 1),
                                1, c_dim, rb_proj)
    out = _attn_layer(bias2d, f1_2, f2_2.T, fts2, bias2.reshape(1, c_dim),
                      1, c_dim, elu=False, row_block=rb_attn)
    return out.reshape(1, n, c_dim)
